# baseline (speedup 1.0000x reference)
import jax
import jax.numpy as jnp
from jax import lax
from jax.experimental import pallas as pl
from jax.experimental.pallas import tpu as pltpu
from jax.experimental.pallas import tpu_sc as plsc

ROWS = 64
N = 8192
TOP_K = 512
L = 16
CH = N // L
NB = 256
NW = 32
R = 2


def _to_key(bits):
    m = lax.shift_right_arithmetic(bits, 31)
    return bits ^ (jnp.bitwise_not(m) & jnp.int32(0x7FFFFFFF))


def _sort_body(in_hbm, sel_hbm, not_hbm,
               v0, v1, ka0, ka1, ia0, ia1, kb0, kb1, ib0, ib1, h0, h1,
               dma_sem):
    wid = lax.axis_index("s") * 2 + lax.axis_index("c")
    lanes = lax.iota(jnp.int32, L)
    g_base = lanes * CH
    ones = jnp.ones((L,), jnp.int32)
    zeros = jnp.zeros((L,), jnp.int32)
    row0 = wid * R

    vals = (v0, v1)
    ka = (ka0, ka1)
    ia = (ia0, ia1)
    kb = (kb0, kb1)
    ib = (ib0, ib1)
    hist = (h0, h1)

    for r in range(R):
        pltpu.sync_copy(in_hbm.at[row0 + r], vals[r])

    def init_step(t, _):
        sl = pl.ds(t * L, L)
        for r in range(R):
            bits = lax.bitcast_convert_type(vals[r][sl] + jnp.float32(0.0),
                                            jnp.int32)
            ka[r][sl] = _to_key(bits)
        return 0
    lax.fori_loop(0, N // L, init_step, 0)

    def do_pass(p, src_key, src_idx, dst_key, dst_idx, first):
        shift = p * 8

        def dig(k):
            if shift:
                k = lax.shift_right_logical(k, shift)
            return k & jnp.int32(0xFF)

        def zero_step(i, _):
            sl = pl.ds(i * L, L)
            for r in range(R):
                hist[r][sl] = zeros
            return 0
        lax.fori_loop(0, NB, zero_step, 0)

        def hist_step(t, _):
            g = g_base + t
            for r in range(R):
                k = plsc.load_gather(src_key[r], [g])
                plsc.addupdate_scatter(hist[r], [dig(k) * L + lanes], ones)
            return 0
        lax.fori_loop(0, CH, hist_step, 0)

        def scan_step(i, carry):
            sl = pl.ds(i * L, L)
            nxt = []
            for r in range(R):
                h = hist[r][sl]
                inc = plsc.cumsum(h)
                hist[r][sl] = inc - h + carry[r]
                nxt.append(carry[r] + inc[L - 1])
            return tuple(nxt)
        lax.fori_loop(0, NB, scan_step, (jnp.int32(0),) * R)

        def scatter_step(t, _):
            g = g_base + t
            for r in range(R):
                k = plsc.load_gather(src_key[r], [g])
                if first:
                    i = g
                else:
                    i = plsc.load_gather(src_idx[r], [g])
                d = dig(k)
                addr = d * L + lanes
                off = plsc.load_gather(hist[r], [addr])
                plsc.store_scatter(dst_key[r], [off], k)
                plsc.store_scatter(dst_idx[r], [off], i)
                plsc.addupdate_scatter(hist[r], [addr], ones)
            return 0
        lax.fori_loop(0, CH, scatter_step, 0)

    do_pass(0, ka, None, kb, ib, True)
    do_pass(1, kb, ib, ka, ia, False)
    do_pass(2, ka, ia, kb, ib, False)
    do_pass(3, kb, ib, ka, ia, False)

    for r in range(R):
        pltpu.sync_copy(ia[r].at[pl.ds(0, TOP_K)], sel_hbm.at[row0 + r])
        pltpu.sync_copy(ia[r].at[pl.ds(TOP_K, N - TOP_K)],
                        not_hbm.at[row0 + r])


@jax.jit
def _run(inputs):
    mesh = plsc.VectorSubcoreMesh(core_axis_name="c", subcore_axis_name="s")
    f = pl.kernel(
        _sort_body,
        out_type=(
            jax.ShapeDtypeStruct((ROWS, TOP_K), jnp.int32),
            jax.ShapeDtypeStruct((ROWS, N - TOP_K), jnp.int32),
        ),
        mesh=mesh,
        scratch_types=[
            pltpu.VMEM((N,), jnp.float32),
            pltpu.VMEM((N,), jnp.float32),
            pltpu.VMEM((N,), jnp.int32),
            pltpu.VMEM((N,), jnp.int32),
            pltpu.VMEM((N,), jnp.int32),
            pltpu.VMEM((N,), jnp.int32),
            pltpu.VMEM((N,), jnp.int32),
            pltpu.VMEM((N,), jnp.int32),
            pltpu.VMEM((N,), jnp.int32),
            pltpu.VMEM((N,), jnp.int32),
            pltpu.VMEM((NB * L,), jnp.int32),
            pltpu.VMEM((NB * L,), jnp.int32),
            pltpu.SemaphoreType.DMA,
        ],
        compiler_params=pltpu.CompilerParams(needs_layout_passes=False),
    )
    return f(inputs)


def kernel(inputs):
    return _run(inputs)


# speedup vs baseline: 1.2634x; 1.2634x over previous
import jax
import jax.numpy as jnp
from jax import lax
from jax.experimental import pallas as pl
from jax.experimental.pallas import tpu as pltpu
from jax.experimental.pallas import tpu_sc as plsc

ROWS = 64
N = 8192
TOP_K = 512
L = 16
CH = N // L
NB = 256
NW = 32
R = 2


def _to_key(bits):
    m = lax.shift_right_arithmetic(bits, 31)
    return bits ^ (jnp.bitwise_not(m) & jnp.int32(0x7FFFFFFF))


def _sort_body(in_hbm, sel_hbm, not_hbm,
               v0, v1, ka0, ka1, ia0, ia1, kb0, kb1, ib0, ib1, h0, h1,
               dma_sem):
    wid = lax.axis_index("s") * 2 + lax.axis_index("c")
    lanes = lax.iota(jnp.int32, L)
    g_base = lanes * CH
    ones = jnp.ones((L,), jnp.int32)
    zeros = jnp.zeros((L,), jnp.int32)
    row0 = wid * R

    vals = (v0, v1)
    ka = (ka0, ka1)
    ia = (ia0, ia1)
    kb = (kb0, kb1)
    ib = (ib0, ib1)
    hist = (h0, h1)

    for r in range(R):
        pltpu.sync_copy(in_hbm.at[row0 + r], vals[r])

    def init_step(t, _):
        sl = pl.ds(t * L, L)
        for r in range(R):
            bits = lax.bitcast_convert_type(vals[r][sl] + jnp.float32(0.0),
                                            jnp.int32)
            ka[r][sl] = _to_key(bits)
        return 0
    lax.fori_loop(0, N // L, init_step, 0)

    def do_pass(p, src_key, src_idx, dst_key, dst_idx, first):
        shift = p * 8

        def dig(k):
            if shift:
                k = lax.shift_right_logical(k, shift)
            return k & jnp.int32(0xFF)

        def zero_step(i, _):
            sl = pl.ds(i * L, L)
            for r in range(R):
                hist[r][sl] = zeros
            return 0
        lax.fori_loop(0, NB, zero_step, 0)

        def hist_step(t, _):
            g = g_base + t
            k = [plsc.load_gather(src_key[r], [g]) for r in range(R)]
            a = [dig(k[r]) * L + lanes for r in range(R)]
            for r in range(R):
                plsc.addupdate_scatter(hist[r], [a[r]], ones)
            return 0
        lax.fori_loop(0, CH, hist_step, 0)

        def scan_step(i, carry):
            sl = pl.ds(i * L, L)
            nxt = []
            for r in range(R):
                h = hist[r][sl]
                inc = plsc.cumsum(h)
                hist[r][sl] = inc - h + carry[r]
                nxt.append(carry[r] + inc[L - 1])
            return tuple(nxt)
        lax.fori_loop(0, NB, scan_step, (jnp.int32(0),) * R)

        def scatter_step(t, _):
            g = g_base + t
            k = [plsc.load_gather(src_key[r], [g]) for r in range(R)]
            if first:
                i = [g for _ in range(R)]
            else:
                i = [plsc.load_gather(src_idx[r], [g]) for r in range(R)]
            a = [dig(k[r]) * L + lanes for r in range(R)]
            off = [plsc.load_gather(hist[r], [a[r]]) for r in range(R)]
            for r in range(R):
                plsc.store_scatter(dst_key[r], [off[r]], k[r])
            for r in range(R):
                plsc.store_scatter(dst_idx[r], [off[r]], i[r])
            for r in range(R):
                plsc.addupdate_scatter(hist[r], [a[r]], ones)
            return 0
        lax.fori_loop(0, CH, scatter_step, 0)

    do_pass(0, ka, None, kb, ib, True)
    do_pass(1, kb, ib, ka, ia, False)
    do_pass(2, ka, ia, kb, ib, False)
    do_pass(3, kb, ib, ka, ia, False)

    for r in range(R):
        pltpu.sync_copy(ia[r].at[pl.ds(0, TOP_K)], sel_hbm.at[row0 + r])
        pltpu.sync_copy(ia[r].at[pl.ds(TOP_K, N - TOP_K)],
                        not_hbm.at[row0 + r])


@jax.jit
def _run(inputs):
    mesh = plsc.VectorSubcoreMesh(core_axis_name="c", subcore_axis_name="s")
    f = pl.kernel(
        _sort_body,
        out_type=(
            jax.ShapeDtypeStruct((ROWS, TOP_K), jnp.int32),
            jax.ShapeDtypeStruct((ROWS, N - TOP_K), jnp.int32),
        ),
        mesh=mesh,
        scratch_types=[
            pltpu.VMEM((N,), jnp.float32),
            pltpu.VMEM((N,), jnp.float32),
            pltpu.VMEM((N,), jnp.int32),
            pltpu.VMEM((N,), jnp.int32),
            pltpu.VMEM((N,), jnp.int32),
            pltpu.VMEM((N,), jnp.int32),
            pltpu.VMEM((N,), jnp.int32),
            pltpu.VMEM((N,), jnp.int32),
            pltpu.VMEM((N,), jnp.int32),
            pltpu.VMEM((N,), jnp.int32),
            pltpu.VMEM((NB * L,), jnp.int32),
            pltpu.VMEM((NB * L,), jnp.int32),
            pltpu.SemaphoreType.DMA,
        ],
        compiler_params=pltpu.CompilerParams(needs_layout_passes=False),
    )
    return f(inputs)


def kernel(inputs):
    return _run(inputs)


# speedup vs baseline: 2.6132x; 2.0683x over previous
import jax
import jax.numpy as jnp
from jax import lax
from jax.experimental import pallas as pl
from jax.experimental.pallas import tpu as pltpu
from jax.experimental.pallas import tpu_sc as plsc

ROWS = 64
N = 8192
TOP_K = 512
L = 16
CH = N // L
NB = 256
NW = 32
R = 2


def _to_key(bits):
    m = lax.shift_right_arithmetic(bits, 31)
    return bits ^ (jnp.bitwise_not(m) & jnp.int32(0x7FFFFFFF))


def _sort_body(in_hbm, sel_hbm, not_hbm,
               v0, v1, ka0, ka1, kb0, kb1, ia0, ia1, ib0, ib1,
               hc0, hc1, hn0, hn1, dma_sem):
    wid = lax.axis_index("s") * 2 + lax.axis_index("c")
    lanes = lax.iota(jnp.int32, L)
    g_base = lanes * CH
    ones = jnp.ones((L,), jnp.int32)
    zeros = jnp.zeros((L,), jnp.int32)
    row0 = wid * R

    vals = (v0, v1)
    ka = (ka0, ka1)
    kb = (kb0, kb1)
    ia = (ia0, ia1)
    ib = (ib0, ib1)
    hc = (hc0, hc1)
    hn = (hn0, hn1)

    for r in range(R):
        pltpu.sync_copy(in_hbm.at[row0 + r], vals[r])

    def zero(hists):
        def step(i, _):
            sl = pl.ds(i * L, L)
            for h in hists:
                h[sl] = zeros
            return 0
        lax.fori_loop(0, NB, step, 0)

    def scan(hists):
        def step(i, carry):
            sl = pl.ds(i * L, L)
            nxt = []
            for r in range(R):
                h = hists[r][sl]
                inc = plsc.cumsum(h)
                hists[r][sl] = inc - h + carry[r]
                nxt.append(carry[r] + inc[L - 1])
            return tuple(nxt)
        lax.fori_loop(0, NB, step, (jnp.int32(0),) * R)

    def addr0(k):
        return (lax.shift_left(k, 4) | lanes) & jnp.int32(0xFFF)

    def addrp(k, shift, low):
        return (lax.shift_right_logical(k, shift - 4) & jnp.int32(0xFF0)) | low

    def s0_step(t, _):
        g = g_base + t
        v = [plsc.load_gather(vals[r], [g]) for r in range(R)]
        k = [_to_key(lax.bitcast_convert_type(v[r] + jnp.float32(0.0),
                                              jnp.int32)) for r in range(R)]
        a = [addr0(k[r]) for r in range(R)]
        sl = pl.ds(t * L, L)
        for r in range(R):
            ka[r][sl] = k[r]
        for r in range(R):
            plsc.addupdate_scatter(hc[r], [a[r]], ones)
        return 0

    def scat(p, ks, is_, kd, id_, hcur, hnxt, first, last):
        shift = 8 * p

        def step(t, _):
            sl = pl.ds(t * L, L)
            g = g_base + t
            k = [ks[r][sl] for r in range(R)]
            if first:
                i = [g for _ in range(R)]
            else:
                i = [is_[r][sl] for r in range(R)]
            if p == 0:
                a = [addr0(k[r]) for r in range(R)]
            else:
                a = [addrp(k[r], shift, lanes) for r in range(R)]
            off = [plsc.load_gather(hcur[r], [a[r]]) for r in range(R)]
            if last:
                for r in range(R):
                    plsc.store_scatter(id_[r], [off[r]], i[r])
                for r in range(R):
                    plsc.addupdate_scatter(hcur[r], [a[r]], ones)
            else:
                own = [lax.shift_right_logical(off[r], 9) for r in range(R)]
                q = [(lax.shift_left(off[r] & jnp.int32(CH - 1), 4)
                      | own[r]) for r in range(R)]
                a2 = [addrp(k[r], shift + 8, own[r]) for r in range(R)]
                for r in range(R):
                    plsc.store_scatter(kd[r], [q[r]], k[r])
                for r in range(R):
                    plsc.store_scatter(id_[r], [q[r]], i[r])
                for r in range(R):
                    plsc.addupdate_scatter(hcur[r], [a[r]], ones)
                for r in range(R):
                    plsc.addupdate_scatter(hnxt[r], [a2[r]], ones)
            return 0
        lax.fori_loop(0, CH, step, 0)

    zero(hc + hn)
    lax.fori_loop(0, CH, s0_step, 0)
    scan(hc)
    scat(0, ka, None, kb, ib, hc, hn, True, False)
    zero(hc)
    scan(hn)
    scat(1, kb, ib, ka, ia, hn, hc, False, False)
    zero(hn)
    scan(hc)
    scat(2, ka, ia, kb, ib, hc, hn, False, False)
    scan(hn)
    scat(3, kb, ib, None, ia, hn, None, False, True)

    for r in range(R):
        pltpu.sync_copy(ia[r].at[pl.ds(0, TOP_K)], sel_hbm.at[row0 + r])
        pltpu.sync_copy(ia[r].at[pl.ds(TOP_K, N - TOP_K)],
                        not_hbm.at[row0 + r])


@jax.jit
def _run(inputs):
    mesh = plsc.VectorSubcoreMesh(core_axis_name="c", subcore_axis_name="s")
    f = pl.kernel(
        _sort_body,
        out_type=(
            jax.ShapeDtypeStruct((ROWS, TOP_K), jnp.int32),
            jax.ShapeDtypeStruct((ROWS, N - TOP_K), jnp.int32),
        ),
        mesh=mesh,
        scratch_types=[
            pltpu.VMEM((N,), jnp.float32),
            pltpu.VMEM((N,), jnp.float32),
            pltpu.VMEM((N,), jnp.int32),
            pltpu.VMEM((N,), jnp.int32),
            pltpu.VMEM((N,), jnp.int32),
            pltpu.VMEM((N,), jnp.int32),
            pltpu.VMEM((N,), jnp.int32),
            pltpu.VMEM((N,), jnp.int32),
            pltpu.VMEM((N,), jnp.int32),
            pltpu.VMEM((N,), jnp.int32),
            pltpu.VMEM((NB * L,), jnp.int32),
            pltpu.VMEM((NB * L,), jnp.int32),
            pltpu.VMEM((NB * L,), jnp.int32),
            pltpu.VMEM((NB * L,), jnp.int32),
            pltpu.SemaphoreType.DMA,
        ],
        compiler_params=pltpu.CompilerParams(needs_layout_passes=False),
    )
    return f(inputs)


def kernel(inputs):
    return _run(inputs)


# speedup vs baseline: 2.6832x; 1.0268x over previous
import jax
import jax.numpy as jnp
from jax import lax
from jax.experimental import pallas as pl
from jax.experimental.pallas import tpu as pltpu
from jax.experimental.pallas import tpu_sc as plsc

ROWS = 64
N = 8192
TOP_K = 512
L = 16
CH = N // L
NB = 256
NW = 32
R = 2


def _to_key(bits):
    m = lax.shift_right_arithmetic(bits, 31)
    return bits ^ (jnp.bitwise_not(m) & jnp.int32(0x7FFFFFFF))


def _sort_body(in_hbm, sel_hbm, not_hbm,
               v0, v1, ka0, ka1, kb0, kb1, ia0, ia1, ib0, ib1,
               hc0, hc1, hn0, hn1, dma_sem):
    wid = lax.axis_index("s") * 2 + lax.axis_index("c")
    lanes = lax.iota(jnp.int32, L)
    g_base = lanes * CH
    ones = jnp.ones((L,), jnp.int32)
    zeros = jnp.zeros((L,), jnp.int32)
    row0 = wid * R

    vals = (v0, v1)
    ka = (ka0, ka1)
    kb = (kb0, kb1)
    ia = (ia0, ia1)
    ib = (ib0, ib1)
    hc = (hc0, hc1)
    hn = (hn0, hn1)

    for r in range(R):
        pltpu.sync_copy(in_hbm.at[row0 + r], vals[r])

    def zero(hists):
        def step(i, _):
            sl = pl.ds(i * L, L)
            for h in hists:
                h[sl] = zeros
            return 0
        lax.fori_loop(0, NB, step, 0)

    def scan(hists, zhists=None):
        def step(i, carry):
            sl = pl.ds(i * L, L)
            nxt = []
            for r in range(R):
                h = hists[r][sl]
                inc = plsc.cumsum(h)
                hists[r][sl] = inc - h + carry[r]
                nxt.append(carry[r] + inc[L - 1])
            if zhists is not None:
                for z in zhists:
                    z[sl] = zeros
            return tuple(nxt)
        lax.fori_loop(0, NB, step, (jnp.int32(0),) * R)

    def addr0(k):
        return (lax.shift_left(k, 4) | lanes) & jnp.int32(0xFFF)

    def addrp(k, shift, low):
        return (lax.shift_right_logical(k, shift - 4) & jnp.int32(0xFF0)) | low

    def s0_step(t, _):
        g = g_base + t
        v = [plsc.load_gather(vals[r], [g]) for r in range(R)]
        k = [_to_key(lax.bitcast_convert_type(v[r] + jnp.float32(0.0),
                                              jnp.int32)) for r in range(R)]
        a = [addr0(k[r]) for r in range(R)]
        sl = pl.ds(t * L, L)
        for r in range(R):
            ka[r][sl] = k[r]
        for r in range(R):
            plsc.addupdate_scatter(hc[r], [a[r]], ones)
        return 0

    def scat(p, ks, is_, kd, id_, hcur, hnxt, first, last):
        shift = 8 * p

        def step(t, _):
            sl = pl.ds(t * L, L)
            g = g_base + t
            k = [ks[r][sl] for r in range(R)]
            if first:
                i = [g for _ in range(R)]
            else:
                i = [is_[r][sl] for r in range(R)]
            if p == 0:
                a = [addr0(k[r]) for r in range(R)]
            else:
                a = [addrp(k[r], shift, lanes) for r in range(R)]
            off = [plsc.load_gather(hcur[r], [a[r]]) for r in range(R)]
            if last:
                for r in range(R):
                    plsc.store_scatter(id_[r], [off[r]], i[r])
                for r in range(R):
                    plsc.addupdate_scatter(hcur[r], [a[r]], ones)
            else:
                own = [lax.shift_right_logical(off[r], 9) for r in range(R)]
                q = [(lax.shift_left(off[r] & jnp.int32(CH - 1), 4)
                      | own[r]) for r in range(R)]
                a2 = [addrp(k[r], shift + 8, own[r]) for r in range(R)]
                for r in range(R):
                    plsc.store_scatter(kd[r], [q[r]], k[r])
                for r in range(R):
                    plsc.store_scatter(id_[r], [q[r]], i[r])
                for r in range(R):
                    plsc.addupdate_scatter(hcur[r], [a[r]], ones)
                for r in range(R):
                    plsc.addupdate_scatter(hnxt[r], [a2[r]], ones)
            return 0
        lax.fori_loop(0, CH, step, 0)

    zero(hc)
    lax.fori_loop(0, CH, s0_step, 0)
    scan(hc, zhists=hn)
    scat(0, ka, None, kb, ib, hc, hn, True, False)
    scan(hn, zhists=hc)
    scat(1, kb, ib, ka, ia, hn, hc, False, False)
    scan(hc, zhists=hn)
    scat(2, ka, ia, kb, ib, hc, hn, False, False)
    scan(hn)
    scat(3, kb, ib, None, ia, hn, None, False, True)

    for r in range(R):
        pltpu.sync_copy(ia[r].at[pl.ds(0, TOP_K)], sel_hbm.at[row0 + r])
        pltpu.sync_copy(ia[r].at[pl.ds(TOP_K, N - TOP_K)],
                        not_hbm.at[row0 + r])


@jax.jit
def _run(inputs):
    mesh = plsc.VectorSubcoreMesh(core_axis_name="c", subcore_axis_name="s")
    f = pl.kernel(
        _sort_body,
        out_type=(
            jax.ShapeDtypeStruct((ROWS, TOP_K), jnp.int32),
            jax.ShapeDtypeStruct((ROWS, N - TOP_K), jnp.int32),
        ),
        mesh=mesh,
        scratch_types=[
            pltpu.VMEM((N,), jnp.float32),
            pltpu.VMEM((N,), jnp.float32),
            pltpu.VMEM((N,), jnp.int32),
            pltpu.VMEM((N,), jnp.int32),
            pltpu.VMEM((N,), jnp.int32),
            pltpu.VMEM((N,), jnp.int32),
            pltpu.VMEM((N,), jnp.int32),
            pltpu.VMEM((N,), jnp.int32),
            pltpu.VMEM((N,), jnp.int32),
            pltpu.VMEM((N,), jnp.int32),
            pltpu.VMEM((NB * L,), jnp.int32),
            pltpu.VMEM((NB * L,), jnp.int32),
            pltpu.VMEM((NB * L,), jnp.int32),
            pltpu.VMEM((NB * L,), jnp.int32),
            pltpu.SemaphoreType.DMA,
        ],
        compiler_params=pltpu.CompilerParams(needs_layout_passes=False),
    )
    return f(inputs)


def kernel(inputs):
    return _run(inputs)


# speedup vs baseline: 2.6980x; 1.0055x over previous
import jax
import jax.numpy as jnp
from jax import lax
from jax.experimental import pallas as pl
from jax.experimental.pallas import tpu as pltpu
from jax.experimental.pallas import tpu_sc as plsc

ROWS = 64
N = 8192
TOP_K = 512
L = 16
CH = N // L
NB = 256
NW = 32
R = 2


def _to_key(bits):
    m = lax.shift_right_arithmetic(bits, 31)
    return bits ^ (jnp.bitwise_not(m) & jnp.int32(0x7FFFFFFF))


def _sort_body(in_hbm, sel_hbm, not_hbm,
               v0, v1, ka0, ka1, kb0, kb1, ia0, ia1, ib0, ib1,
               hc0, hc1, hn0, hn1, dma_sem):
    wid = lax.axis_index("s") * 2 + lax.axis_index("c")
    lanes = lax.iota(jnp.int32, L)
    g_base = lanes * CH
    ones = jnp.ones((L,), jnp.int32)
    zeros = jnp.zeros((L,), jnp.int32)
    row0 = wid * R

    vals = (v0, v1)
    ka = (ka0, ka1)
    kb = (kb0, kb1)
    ia = (ia0, ia1)
    ib = (ib0, ib1)
    hc = (hc0, hc1)
    hn = (hn0, hn1)

    for r in range(R):
        pltpu.sync_copy(in_hbm.at[row0 + r], vals[r])

    def zero(hists):
        def step(i, _):
            sl = pl.ds(i * L, L)
            for h in hists:
                h[sl] = zeros
            return 0
        lax.fori_loop(0, NB, step, 0)

    def scan(hists, zhists=None):
        def step(i, carry):
            sl = pl.ds(i * L, L)
            nxt = []
            for r in range(R):
                h = hists[r][sl]
                inc = plsc.cumsum(h)
                hists[r][sl] = inc - h + carry[r]
                nxt.append(carry[r] + inc[L - 1])
            if zhists is not None:
                for z in zhists:
                    z[sl] = zeros
            return tuple(nxt)
        lax.fori_loop(0, NB, step, (jnp.int32(0),) * R)

    def addr0(k):
        return (lax.shift_left(k, 4) | lanes) & jnp.int32(0xFFF)

    def addrp(k, shift, low):
        return (lax.shift_right_logical(k, shift - 4) & jnp.int32(0xFF0)) | low

    def s0_step(t, _):
        g = g_base + t
        v = [plsc.load_gather(vals[r], [g]) for r in range(R)]
        k = [_to_key(lax.bitcast_convert_type(v[r] + jnp.float32(0.0),
                                              jnp.int32)) for r in range(R)]
        a = [addr0(k[r]) for r in range(R)]
        sl = pl.ds(t * L, L)
        for r in range(R):
            ka[r][sl] = k[r]
        for r in range(R):
            plsc.addupdate_scatter(hc[r], [a[r]], ones)
        return 0

    def scat(p, ks, is_, kd, id_, hcur, hnxt, first, last):
        shift = 8 * p

        def step(t, _):
            sl = pl.ds(t * L, L)
            g = g_base + t
            k = [ks[r][sl] for r in range(R)]
            if first:
                i = [g for _ in range(R)]
            else:
                i = [is_[r][sl] for r in range(R)]
            if p == 0:
                a = [addr0(k[r]) for r in range(R)]
            else:
                a = [addrp(k[r], shift, lanes) for r in range(R)]
            off = [plsc.load_gather(hcur[r], [a[r]]) for r in range(R)]
            if last:
                for r in range(R):
                    plsc.store_scatter(id_[r], [off[r]], i[r])
                for r in range(R):
                    plsc.addupdate_scatter(hcur[r], [a[r]], ones)
            else:
                own = [lax.shift_right_logical(off[r], 9) for r in range(R)]
                q = [(lax.shift_left(off[r] & jnp.int32(CH - 1), 4)
                      | own[r]) for r in range(R)]
                a2 = [addrp(k[r], shift + 8, own[r]) for r in range(R)]
                for r in range(R):
                    plsc.store_scatter(kd[r], [q[r]], k[r])
                for r in range(R):
                    plsc.store_scatter(id_[r], [q[r]], i[r])
                for r in range(R):
                    plsc.addupdate_scatter(hcur[r], [a[r]], ones)
                for r in range(R):
                    plsc.addupdate_scatter(hnxt[r], [a2[r]], ones)
            return 0
        lax.fori_loop(0, CH, step, 0, unroll=2)

    zero(hc)
    lax.fori_loop(0, CH, s0_step, 0, unroll=2)
    scan(hc, zhists=hn)
    scat(0, ka, None, kb, ib, hc, hn, True, False)
    scan(hn, zhists=hc)
    scat(1, kb, ib, ka, ia, hn, hc, False, False)
    scan(hc, zhists=hn)
    scat(2, ka, ia, kb, ib, hc, hn, False, False)
    scan(hn)
    scat(3, kb, ib, None, ia, hn, None, False, True)

    for r in range(R):
        pltpu.sync_copy(ia[r].at[pl.ds(0, TOP_K)], sel_hbm.at[row0 + r])
        pltpu.sync_copy(ia[r].at[pl.ds(TOP_K, N - TOP_K)],
                        not_hbm.at[row0 + r])


@jax.jit
def _run(inputs):
    mesh = plsc.VectorSubcoreMesh(core_axis_name="c", subcore_axis_name="s")
    f = pl.kernel(
        _sort_body,
        out_type=(
            jax.ShapeDtypeStruct((ROWS, TOP_K), jnp.int32),
            jax.ShapeDtypeStruct((ROWS, N - TOP_K), jnp.int32),
        ),
        mesh=mesh,
        scratch_types=[
            pltpu.VMEM((N,), jnp.float32),
            pltpu.VMEM((N,), jnp.float32),
            pltpu.VMEM((N,), jnp.int32),
            pltpu.VMEM((N,), jnp.int32),
            pltpu.VMEM((N,), jnp.int32),
            pltpu.VMEM((N,), jnp.int32),
            pltpu.VMEM((N,), jnp.int32),
            pltpu.VMEM((N,), jnp.int32),
            pltpu.VMEM((N,), jnp.int32),
            pltpu.VMEM((N,), jnp.int32),
            pltpu.VMEM((NB * L,), jnp.int32),
            pltpu.VMEM((NB * L,), jnp.int32),
            pltpu.VMEM((NB * L,), jnp.int32),
            pltpu.VMEM((NB * L,), jnp.int32),
            pltpu.SemaphoreType.DMA,
        ],
        compiler_params=pltpu.CompilerParams(needs_layout_passes=False),
    )
    return f(inputs)


def kernel(inputs):
    return _run(inputs)


# speedup vs baseline: 2.7365x; 1.0143x over previous
import jax
import jax.numpy as jnp
from jax import lax
from jax.experimental import pallas as pl
from jax.experimental.pallas import tpu as pltpu
from jax.experimental.pallas import tpu_sc as plsc

ROWS = 64
N = 8192
TOP_K = 512
L = 16
CH = N // L
NB = 256
NW = 32
R = 2


def _to_key(bits):
    m = lax.shift_right_arithmetic(bits, 31)
    return bits ^ (jnp.bitwise_not(m) & jnp.int32(0x7FFFFFFF))


def _sort_body(in_hbm, sel_hbm, not_hbm,
               v0, v1, ka0, ka1, kb0, kb1, ia0, ia1, ib0, ib1,
               hc0, hc1, hn0, hn1, dma_sem):
    wid = lax.axis_index("s") * 2 + lax.axis_index("c")
    lanes = lax.iota(jnp.int32, L)
    g_base = lanes * CH
    ones = jnp.ones((L,), jnp.int32)
    zeros = jnp.zeros((L,), jnp.int32)
    row0 = wid * R

    vals = (v0, v1)
    ka = (ka0, ka1)
    kb = (kb0, kb1)
    ia = (ia0, ia1)
    ib = (ib0, ib1)
    hc = (hc0, hc1)
    hn = (hn0, hn1)

    for r in range(R):
        pltpu.sync_copy(in_hbm.at[row0 + r], vals[r])

    def zero(hists):
        def step(i, _):
            sl = pl.ds(i * L, L)
            for h in hists:
                h[sl] = zeros
            return 0
        lax.fori_loop(0, NB, step, 0)

    def scan(hists, zhists=None):
        def step(i, carry):
            sl = pl.ds(i * L, L)
            nxt = []
            for r in range(R):
                h = hists[r][sl]
                inc = plsc.cumsum(h)
                hists[r][sl] = inc - h + carry[r]
                nxt.append(carry[r] + inc[L - 1])
            if zhists is not None:
                for z in zhists:
                    z[sl] = zeros
            return tuple(nxt)
        lax.fori_loop(0, NB, step, (jnp.int32(0),) * R)

    def addr0(k):
        return (lax.shift_left(k, 4) | lanes) & jnp.int32(0xFFF)

    def addrp(k, shift, low):
        return (lax.shift_right_logical(k, shift - 4) & jnp.int32(0xFF0)) | low

    def s0_step(t, _):
        g = g_base + t
        v = [plsc.load_gather(vals[r], [g]) for r in range(R)]
        k = [_to_key(lax.bitcast_convert_type(v[r] + jnp.float32(0.0),
                                              jnp.int32)) for r in range(R)]
        a = [addr0(k[r]) for r in range(R)]
        sl = pl.ds(t * L, L)
        for r in range(R):
            ka[r][sl] = k[r]
        for r in range(R):
            plsc.addupdate_scatter(hc[r], [a[r]], ones)
        return 0

    def dest(off):
        own = lax.shift_right_logical(off, 9)
        q = lax.shift_left(off & jnp.int32(CH - 1), 4) | own
        return own, q

    def scat0():
        def step(t, _):
            sl = pl.ds(t * L, L)
            g = g_base + t
            k = [ka[r][sl] for r in range(R)]
            a = [addr0(k[r]) for r in range(R)]
            off = [plsc.load_gather(hc[r], [a[r]]) for r in range(R)]
            oq = [dest(off[r]) for r in range(R)]
            a2 = [addrp(k[r], 8, oq[r][0]) for r in range(R)]
            for r in range(R):
                plsc.store_scatter(kb[r], [oq[r][1]], k[r])
            for r in range(R):
                plsc.store_scatter(ib[r], [oq[r][1]], g)
            for r in range(R):
                plsc.addupdate_scatter(hc[r], [a[r]], ones)
            for r in range(R):
                plsc.addupdate_scatter(hn[r], [a2[r]], ones)
            return 0
        lax.fori_loop(0, CH, step, 0, unroll=2)

    def scat1():
        def step(t, _):
            sl = pl.ds(t * L, L)
            k = [kb[r][sl] for r in range(R)]
            i = [ib[r][sl] for r in range(R)]
            a = [addrp(k[r], 8, lanes) for r in range(R)]
            off = [plsc.load_gather(hn[r], [a[r]]) for r in range(R)]
            oq = [dest(off[r]) for r in range(R)]
            a2 = [addrp(k[r], 16, oq[r][0]) for r in range(R)]
            pk = [lax.shift_left(lax.shift_right_logical(k[r], 16), 13)
                  | i[r] for r in range(R)]
            for r in range(R):
                plsc.store_scatter(ka[r], [oq[r][1]], pk[r])
            for r in range(R):
                plsc.addupdate_scatter(hn[r], [a[r]], ones)
            for r in range(R):
                plsc.addupdate_scatter(hc[r], [a2[r]], ones)
            return 0
        lax.fori_loop(0, CH, step, 0, unroll=2)

    def addr_pk(p_, sh, low):
        return (lax.shift_right_logical(p_, sh) & jnp.int32(0xFF0)) | low

    def scat2():
        def step(t, _):
            sl = pl.ds(t * L, L)
            p_ = [ka[r][sl] for r in range(R)]
            a = [addr_pk(p_[r], 9, lanes) for r in range(R)]
            off = [plsc.load_gather(hc[r], [a[r]]) for r in range(R)]
            oq = [dest(off[r]) for r in range(R)]
            a2 = [addr_pk(p_[r], 17, oq[r][0]) for r in range(R)]
            for r in range(R):
                plsc.store_scatter(kb[r], [oq[r][1]], p_[r])
            for r in range(R):
                plsc.addupdate_scatter(hc[r], [a[r]], ones)
            for r in range(R):
                plsc.addupdate_scatter(hn[r], [a2[r]], ones)
            return 0
        lax.fori_loop(0, CH, step, 0, unroll=2)

    def scat3():
        def step(t, _):
            sl = pl.ds(t * L, L)
            p_ = [kb[r][sl] for r in range(R)]
            a = [addr_pk(p_[r], 17, lanes) for r in range(R)]
            off = [plsc.load_gather(hn[r], [a[r]]) for r in range(R)]
            for r in range(R):
                plsc.store_scatter(ia[r], [off[r]],
                                   p_[r] & jnp.int32(0x1FFF))
            for r in range(R):
                plsc.addupdate_scatter(hn[r], [a[r]], ones)
            return 0
        lax.fori_loop(0, CH, step, 0, unroll=2)

    zero(hc)
    lax.fori_loop(0, CH, s0_step, 0, unroll=2)
    scan(hc, zhists=hn)
    scat0()
    scan(hn, zhists=hc)
    scat1()
    scan(hc, zhists=hn)
    scat2()
    scan(hn)
    scat3()

    for r in range(R):
        pltpu.sync_copy(ia[r].at[pl.ds(0, TOP_K)], sel_hbm.at[row0 + r])
        pltpu.sync_copy(ia[r].at[pl.ds(TOP_K, N - TOP_K)],
                        not_hbm.at[row0 + r])


@jax.jit
def _run(inputs):
    mesh = plsc.VectorSubcoreMesh(core_axis_name="c", subcore_axis_name="s")
    f = pl.kernel(
        _sort_body,
        out_type=(
            jax.ShapeDtypeStruct((ROWS, TOP_K), jnp.int32),
            jax.ShapeDtypeStruct((ROWS, N - TOP_K), jnp.int32),
        ),
        mesh=mesh,
        scratch_types=[
            pltpu.VMEM((N,), jnp.float32),
            pltpu.VMEM((N,), jnp.float32),
            pltpu.VMEM((N,), jnp.int32),
            pltpu.VMEM((N,), jnp.int32),
            pltpu.VMEM((N,), jnp.int32),
            pltpu.VMEM((N,), jnp.int32),
            pltpu.VMEM((N,), jnp.int32),
            pltpu.VMEM((N,), jnp.int32),
            pltpu.VMEM((N,), jnp.int32),
            pltpu.VMEM((N,), jnp.int32),
            pltpu.VMEM((NB * L,), jnp.int32),
            pltpu.VMEM((NB * L,), jnp.int32),
            pltpu.VMEM((NB * L,), jnp.int32),
            pltpu.VMEM((NB * L,), jnp.int32),
            pltpu.SemaphoreType.DMA,
        ],
        compiler_params=pltpu.CompilerParams(needs_layout_passes=False),
    )
    return f(inputs)


def kernel(inputs):
    return _run(inputs)


# speedup vs baseline: 3.3465x; 1.2229x over previous
import jax
import jax.numpy as jnp
from jax import lax
from jax.experimental import pallas as pl
from jax.experimental.pallas import tpu as pltpu
from jax.experimental.pallas import tpu_sc as plsc

ROWS = 64
N = 8192
TOP_K = 512
L = 16
CH = N // L
HH = CH // 2
NB = 256
HB = NB * L
NW = 32
R = 2


def _to_key(bits):
    m = lax.shift_right_arithmetic(bits, 31)
    return bits ^ (jnp.bitwise_not(m) & jnp.int32(0x7FFFFFFF))


def _sort_body(in_hbm, sel_hbm, not_hbm,
               v0, v1, ka0, ka1, kb0, kb1, ia0, ia1, ib0, ib1,
               hc0, hc1, hn0, hn1, dma_sem):
    wid = lax.axis_index("s") * 2 + lax.axis_index("c")
    lanes = lax.iota(jnp.int32, L)
    g_base = lanes * CH
    ones = jnp.ones((L,), jnp.int32)
    zeros = jnp.zeros((L,), jnp.int32)
    row0 = wid * R

    vals = (v0, v1)
    ka = (ka0, ka1)
    kb = (kb0, kb1)
    ia = (ia0, ia1)
    ib = (ib0, ib1)
    hc = (hc0, hc1)
    hn = (hn0, hn1)
    H = 2

    for r in range(R):
        pltpu.sync_copy(in_hbm.at[row0 + r], vals[r])

    def zero(hists):
        def step(i, _):
            for h in hists:
                for x in range(H):
                    h[pl.ds(i * L + x * HB, L)] = zeros
            return 0
        lax.fori_loop(0, NB, step, 0)

    def scan(hists, zhists=None):
        def step(i, carry):
            sl = pl.ds(i * L, L)
            slB = pl.ds(i * L + HB, L)
            nxt = []
            for r in range(R):
                hA = hists[r][sl]
                hB = hists[r][slB]
                s = hA + hB
                inc = plsc.cumsum(s)
                e = inc - s + carry[r]
                hists[r][sl] = e
                hists[r][slB] = e + hA
                nxt.append(carry[r] + inc[L - 1])
            if zhists is not None:
                for z in zhists:
                    z[sl] = zeros
                    z[slB] = zeros
            return tuple(nxt)
        lax.fori_loop(0, NB, step, (jnp.int32(0),) * R)

    def addr0(k):
        return (lax.shift_left(k, 4) | lanes) & jnp.int32(0xFFF)

    def addrp(k, shift, low):
        return (lax.shift_right_logical(k, shift - 4) & jnp.int32(0xFF0)) | low

    def halfbit(off):
        return lax.shift_left(off & jnp.int32(HH), 4)

    def dest(off):
        own = lax.shift_right_logical(off, 9)
        q = lax.shift_left(off & jnp.int32(CH - 1), 4) | own
        return own, q

    RH = tuple((r, x) for r in range(R) for x in range(H))

    def s0_step(t, _):
        g = [g_base + t + x * HH for (r, x) in RH]
        v = [plsc.load_gather(vals[r], [g[j]]) for j, (r, x) in enumerate(RH)]
        k = [_to_key(lax.bitcast_convert_type(vj + jnp.float32(0.0),
                                              jnp.int32)) for vj in v]
        a = [addr0(k[j]) + x * HB for j, (r, x) in enumerate(RH)]
        for j, (r, x) in enumerate(RH):
            ka[r][pl.ds((t + x * HH) * L, L)] = k[j]
        for j, (r, x) in enumerate(RH):
            plsc.addupdate_scatter(hc[r], [a[j]], ones)
        return 0

    def scat0():
        def step(t, _):
            sl = [pl.ds((t + x * HH) * L, L) for (r, x) in RH]
            g = [g_base + t + x * HH for (r, x) in RH]
            k = [ka[r][sl[j]] for j, (r, x) in enumerate(RH)]
            a = [addr0(k[j]) + x * HB for j, (r, x) in enumerate(RH)]
            off = [plsc.load_gather(hc[r], [a[j]])
                   for j, (r, x) in enumerate(RH)]
            oq = [dest(o) for o in off]
            a2 = [addrp(k[j], 8, oq[j][0]) | halfbit(off[j])
                  for j in range(len(RH))]
            for j, (r, x) in enumerate(RH):
                plsc.store_scatter(kb[r], [oq[j][1]], k[j])
            for j, (r, x) in enumerate(RH):
                plsc.store_scatter(ib[r], [oq[j][1]], g[j])
            for j, (r, x) in enumerate(RH):
                plsc.addupdate_scatter(hc[r], [a[j]], ones)
            for j, (r, x) in enumerate(RH):
                plsc.addupdate_scatter(hn[r], [a2[j]], ones)
            return 0
        lax.fori_loop(0, HH, step, 0)

    def scat1():
        def step(t, _):
            sl = [pl.ds((t + x * HH) * L, L) for (r, x) in RH]
            k = [kb[r][sl[j]] for j, (r, x) in enumerate(RH)]
            i = [ib[r][sl[j]] for j, (r, x) in enumerate(RH)]
            a = [addrp(k[j], 8, lanes) + x * HB
                 for j, (r, x) in enumerate(RH)]
            off = [plsc.load_gather(hn[r], [a[j]])
                   for j, (r, x) in enumerate(RH)]
            oq = [dest(o) for o in off]
            a2 = [addrp(k[j], 16, oq[j][0]) | halfbit(off[j])
                  for j in range(len(RH))]
            pk = [lax.shift_left(lax.shift_right_logical(k[j], 16), 13)
                  | i[j] for j in range(len(RH))]
            for j, (r, x) in enumerate(RH):
                plsc.store_scatter(ka[r], [oq[j][1]], pk[j])
            for j, (r, x) in enumerate(RH):
                plsc.addupdate_scatter(hn[r], [a[j]], ones)
            for j, (r, x) in enumerate(RH):
                plsc.addupdate_scatter(hc[r], [a2[j]], ones)
            return 0
        lax.fori_loop(0, HH, step, 0)

    def addr_pk(p_, sh, low):
        return (lax.shift_right_logical(p_, sh) & jnp.int32(0xFF0)) | low

    def scat2():
        def step(t, _):
            sl = [pl.ds((t + x * HH) * L, L) for (r, x) in RH]
            p_ = [ka[r][sl[j]] for j, (r, x) in enumerate(RH)]
            a = [addr_pk(p_[j], 9, lanes) + x * HB
                 for j, (r, x) in enumerate(RH)]
            off = [plsc.load_gather(hc[r], [a[j]])
                   for j, (r, x) in enumerate(RH)]
            oq = [dest(o) for o in off]
            a2 = [addr_pk(p_[j], 17, oq[j][0]) | halfbit(off[j])
                  for j in range(len(RH))]
            for j, (r, x) in enumerate(RH):
                plsc.store_scatter(kb[r], [oq[j][1]], p_[j])
            for j, (r, x) in enumerate(RH):
                plsc.addupdate_scatter(hc[r], [a[j]], ones)
            for j, (r, x) in enumerate(RH):
                plsc.addupdate_scatter(hn[r], [a2[j]], ones)
            return 0
        lax.fori_loop(0, HH, step, 0)

    def scat3():
        def step(t, _):
            sl = [pl.ds((t + x * HH) * L, L) for (r, x) in RH]
            p_ = [kb[r][sl[j]] for j, (r, x) in enumerate(RH)]
            a = [addr_pk(p_[j], 17, lanes) + x * HB
                 for j, (r, x) in enumerate(RH)]
            off = [plsc.load_gather(hn[r], [a[j]])
                   for j, (r, x) in enumerate(RH)]
            for j, (r, x) in enumerate(RH):
                plsc.store_scatter(ia[r], [off[j]],
                                   p_[j] & jnp.int32(0x1FFF))
            for j, (r, x) in enumerate(RH):
                plsc.addupdate_scatter(hn[r], [a[j]], ones)
            return 0
        lax.fori_loop(0, HH, step, 0)

    zero(hc)
    lax.fori_loop(0, HH, s0_step, 0)
    scan(hc, zhists=hn)
    scat0()
    scan(hn, zhists=hc)
    scat1()
    scan(hc, zhists=hn)
    scat2()
    scan(hn)
    scat3()

    for r in range(R):
        pltpu.sync_copy(ia[r].at[pl.ds(0, TOP_K)], sel_hbm.at[row0 + r])
        pltpu.sync_copy(ia[r].at[pl.ds(TOP_K, N - TOP_K)],
                        not_hbm.at[row0 + r])


@jax.jit
def _run(inputs):
    mesh = plsc.VectorSubcoreMesh(core_axis_name="c", subcore_axis_name="s")
    f = pl.kernel(
        _sort_body,
        out_type=(
            jax.ShapeDtypeStruct((ROWS, TOP_K), jnp.int32),
            jax.ShapeDtypeStruct((ROWS, N - TOP_K), jnp.int32),
        ),
        mesh=mesh,
        scratch_types=[
            pltpu.VMEM((N,), jnp.float32),
            pltpu.VMEM((N,), jnp.float32),
            pltpu.VMEM((N,), jnp.int32),
            pltpu.VMEM((N,), jnp.int32),
            pltpu.VMEM((N,), jnp.int32),
            pltpu.VMEM((N,), jnp.int32),
            pltpu.VMEM((N,), jnp.int32),
            pltpu.VMEM((N,), jnp.int32),
            pltpu.VMEM((N,), jnp.int32),
            pltpu.VMEM((N,), jnp.int32),
            pltpu.VMEM((2 * NB * L,), jnp.int32),
            pltpu.VMEM((2 * NB * L,), jnp.int32),
            pltpu.VMEM((2 * NB * L,), jnp.int32),
            pltpu.VMEM((2 * NB * L,), jnp.int32),
            pltpu.SemaphoreType.DMA,
        ],
        compiler_params=pltpu.CompilerParams(needs_layout_passes=False),
    )
    return f(inputs)


def kernel(inputs):
    return _run(inputs)


# speedup vs baseline: 3.6388x; 1.0874x over previous
import jax
import jax.numpy as jnp
from jax import lax
from jax.experimental import pallas as pl
from jax.experimental.pallas import tpu as pltpu
from jax.experimental.pallas import tpu_sc as plsc

ROWS = 64
N = 8192
TOP_K = 512
L = 16
CH = N // L
H = 4
HH = CH // H
HSH = 7
NB = 256
HB = NB * L
NW = 32
R = 2


def _to_key(bits):
    m = lax.shift_right_arithmetic(bits, 31)
    return bits ^ (jnp.bitwise_not(m) & jnp.int32(0x7FFFFFFF))


def _sort_body(in_hbm, sel_hbm, not_hbm,
               v0, v1, ka0, ka1, kb0, kb1, ib0, ib1,
               hc0, hc1, hn0, hn1, dma_sem):
    wid = lax.axis_index("s") * 2 + lax.axis_index("c")
    lanes = lax.iota(jnp.int32, L)
    g_base = lanes * CH
    ones = jnp.ones((L,), jnp.int32)
    zeros = jnp.zeros((L,), jnp.int32)
    row0 = wid * R

    vals = (v0, v1)
    ka = (ka0, ka1)
    kb = (kb0, kb1)
    ib = (ib0, ib1)
    hc = (hc0, hc1)
    hn = (hn0, hn1)

    for r in range(R):
        pltpu.sync_copy(in_hbm.at[row0 + r], vals[r])

    def zero(hists):
        def step(i, _):
            for h in hists:
                for x in range(H):
                    h[pl.ds(i * L + x * HB, L)] = zeros
            return 0
        lax.fori_loop(0, NB, step, 0)

    def scan(hists, zhists=None):
        def step(i, carry):
            sls = [pl.ds(i * L + x * HB, L) for x in range(H)]
            nxt = []
            for r in range(R):
                hs = [hists[r][s_] for s_ in sls]
                s = hs[0]
                for x in range(1, H):
                    s = s + hs[x]
                inc = plsc.cumsum(s)
                b = inc - s + carry[r]
                for x in range(H):
                    hists[r][sls[x]] = b
                    if x + 1 < H:
                        b = b + hs[x]
                nxt.append(carry[r] + inc[L - 1])
            if zhists is not None:
                for z in zhists:
                    for s_ in sls:
                        z[s_] = zeros
            return tuple(nxt)
        lax.fori_loop(0, NB, step, (jnp.int32(0),) * R)

    def addr0(k):
        return (lax.shift_left(k, 4) | lanes) & jnp.int32(0xFFF)

    def addrp(k, shift, low):
        return (lax.shift_right_logical(k, shift - 4) & jnp.int32(0xFF0)) | low

    def halfbit(off):
        return lax.shift_left(
            lax.shift_right_logical(off, HSH) & jnp.int32(H - 1), 12)

    def dest(off):
        own = lax.shift_right_logical(off, 9)
        q = lax.shift_left(off & jnp.int32(CH - 1), 4) | own
        return own, q

    RH = tuple((r, x) for r in range(R) for x in range(H))

    def s0_step(t, _):
        g = [g_base + t + x * HH for (r, x) in RH]
        v = [plsc.load_gather(vals[r], [g[j]]) for j, (r, x) in enumerate(RH)]
        k = [_to_key(lax.bitcast_convert_type(vj + jnp.float32(0.0),
                                              jnp.int32)) for vj in v]
        a = [addr0(k[j]) + x * HB for j, (r, x) in enumerate(RH)]
        for j, (r, x) in enumerate(RH):
            ka[r][pl.ds((t + x * HH) * L, L)] = k[j]
        for j, (r, x) in enumerate(RH):
            plsc.addupdate_scatter(hc[r], [a[j]], ones)
        return 0

    def scat0():
        def step(t, _):
            sl = [pl.ds((t + x * HH) * L, L) for (r, x) in RH]
            g = [g_base + t + x * HH for (r, x) in RH]
            k = [ka[r][sl[j]] for j, (r, x) in enumerate(RH)]
            a = [addr0(k[j]) + x * HB for j, (r, x) in enumerate(RH)]
            off = [plsc.load_gather(hc[r], [a[j]])
                   for j, (r, x) in enumerate(RH)]
            oq = [dest(o) for o in off]
            a2 = [addrp(k[j], 8, oq[j][0]) | halfbit(off[j])
                  for j in range(len(RH))]
            for j, (r, x) in enumerate(RH):
                plsc.store_scatter(kb[r], [oq[j][1]], k[j])
            for j, (r, x) in enumerate(RH):
                plsc.store_scatter(ib[r], [oq[j][1]], g[j])
            for j, (r, x) in enumerate(RH):
                plsc.addupdate_scatter(hc[r], [a[j]], ones)
            for j, (r, x) in enumerate(RH):
                plsc.addupdate_scatter(hn[r], [a2[j]], ones)
            return 0
        lax.fori_loop(0, HH, step, 0)

    def scat1():
        def step(t, _):
            sl = [pl.ds((t + x * HH) * L, L) for (r, x) in RH]
            k = [kb[r][sl[j]] for j, (r, x) in enumerate(RH)]
            i = [ib[r][sl[j]] for j, (r, x) in enumerate(RH)]
            a = [addrp(k[j], 8, lanes) + x * HB
                 for j, (r, x) in enumerate(RH)]
            off = [plsc.load_gather(hn[r], [a[j]])
                   for j, (r, x) in enumerate(RH)]
            oq = [dest(o) for o in off]
            a2 = [addrp(k[j], 16, oq[j][0]) | halfbit(off[j])
                  for j in range(len(RH))]
            pk = [lax.shift_left(lax.shift_right_logical(k[j], 16), 13)
                  | i[j] for j in range(len(RH))]
            for j, (r, x) in enumerate(RH):
                plsc.store_scatter(ka[r], [oq[j][1]], pk[j])
            for j, (r, x) in enumerate(RH):
                plsc.addupdate_scatter(hn[r], [a[j]], ones)
            for j, (r, x) in enumerate(RH):
                plsc.addupdate_scatter(hc[r], [a2[j]], ones)
            return 0
        lax.fori_loop(0, HH, step, 0)

    def addr_pk(p_, sh, low):
        return (lax.shift_right_logical(p_, sh) & jnp.int32(0xFF0)) | low

    def scat2():
        def step(t, _):
            sl = [pl.ds((t + x * HH) * L, L) for (r, x) in RH]
            p_ = [ka[r][sl[j]] for j, (r, x) in enumerate(RH)]
            a = [addr_pk(p_[j], 9, lanes) + x * HB
                 for j, (r, x) in enumerate(RH)]
            off = [plsc.load_gather(hc[r], [a[j]])
                   for j, (r, x) in enumerate(RH)]
            oq = [dest(o) for o in off]
            a2 = [addr_pk(p_[j], 17, oq[j][0]) | halfbit(off[j])
                  for j in range(len(RH))]
            for j, (r, x) in enumerate(RH):
                plsc.store_scatter(kb[r], [oq[j][1]], p_[j])
            for j, (r, x) in enumerate(RH):
                plsc.addupdate_scatter(hc[r], [a[j]], ones)
            for j, (r, x) in enumerate(RH):
                plsc.addupdate_scatter(hn[r], [a2[j]], ones)
            return 0
        lax.fori_loop(0, HH, step, 0)

    def scat3():
        def step(t, _):
            sl = [pl.ds((t + x * HH) * L, L) for (r, x) in RH]
            p_ = [kb[r][sl[j]] for j, (r, x) in enumerate(RH)]
            a = [addr_pk(p_[j], 17, lanes) + x * HB
                 for j, (r, x) in enumerate(RH)]
            off = [plsc.load_gather(hn[r], [a[j]])
                   for j, (r, x) in enumerate(RH)]
            for j, (r, x) in enumerate(RH):
                plsc.store_scatter(ib[r], [off[j]],
                                   p_[j] & jnp.int32(0x1FFF))
            for j, (r, x) in enumerate(RH):
                plsc.addupdate_scatter(hn[r], [a[j]], ones)
            return 0
        lax.fori_loop(0, HH, step, 0)

    zero(hc)
    lax.fori_loop(0, HH, s0_step, 0)
    scan(hc, zhists=hn)
    scat0()
    scan(hn, zhists=hc)
    scat1()
    scan(hc, zhists=hn)
    scat2()
    scan(hn)
    scat3()

    for r in range(R):
        pltpu.sync_copy(ib[r].at[pl.ds(0, TOP_K)], sel_hbm.at[row0 + r])
        pltpu.sync_copy(ib[r].at[pl.ds(TOP_K, N - TOP_K)],
                        not_hbm.at[row0 + r])


@jax.jit
def _run(inputs):
    mesh = plsc.VectorSubcoreMesh(core_axis_name="c", subcore_axis_name="s")
    f = pl.kernel(
        _sort_body,
        out_type=(
            jax.ShapeDtypeStruct((ROWS, TOP_K), jnp.int32),
            jax.ShapeDtypeStruct((ROWS, N - TOP_K), jnp.int32),
        ),
        mesh=mesh,
        scratch_types=[
            pltpu.VMEM((N,), jnp.float32),
            pltpu.VMEM((N,), jnp.float32),
            pltpu.VMEM((N,), jnp.int32),
            pltpu.VMEM((N,), jnp.int32),
            pltpu.VMEM((N,), jnp.int32),
            pltpu.VMEM((N,), jnp.int32),
            pltpu.VMEM((N,), jnp.int32),
            pltpu.VMEM((N,), jnp.int32),
            pltpu.VMEM((H * NB * L,), jnp.int32),
            pltpu.VMEM((H * NB * L,), jnp.int32),
            pltpu.VMEM((H * NB * L,), jnp.int32),
            pltpu.VMEM((H * NB * L,), jnp.int32),
            pltpu.SemaphoreType.DMA,
        ],
        compiler_params=pltpu.CompilerParams(needs_layout_passes=False),
    )
    return f(inputs)


def kernel(inputs):
    return _run(inputs)
